# baseline (speedup 1.0000x reference)
import functools
import math

import jax
import jax.numpy as jnp
from jax import lax
from jax.experimental import pallas as pl
from jax.experimental.pallas import tpu as pltpu
from jax.experimental.pallas import tpu_sc as plsc

VOCAB = 1000000
EMBED = 64
LANES = 16
NUM_CORES = 2
NUM_SUBCORES = 16
NUM_WORKERS = NUM_CORES * NUM_SUBCORES
CHUNK = 128
NBUF = 4
SCALE = math.sqrt(EMBED)


def _emb_lookup(table, idx3):
    nw, n_chunks, chunk = idx3.shape
    per_w = n_chunks * chunk
    n = nw * per_w
    n_outer = n_chunks // NBUF

    mesh = plsc.VectorSubcoreMesh(core_axis_name="c", subcore_axis_name="s")

    @functools.partial(
        pl.kernel,
        mesh=mesh,
        out_type=jax.ShapeDtypeStruct((n, EMBED), jnp.float32),
        scratch_types=(
            [pltpu.VMEM((n_chunks, chunk), jnp.int32)]
            + [pltpu.VMEM((chunk, EMBED), jnp.float32) for _ in range(2 * NBUF)]
            + [pltpu.SemaphoreType.DMA for _ in range(2 * NBUF)]
        ),
        compiler_params=pltpu.CompilerParams(use_tc_tiling_on_sc=False),
    )
    def k(table_hbm, idx_hbm, out_hbm, idx_v, *rest):
        gbufs = rest[0:NBUF]
        sbufs = rest[NBUF : 2 * NBUF]
        gsems = rest[2 * NBUF : 3 * NBUF]
        ssems = rest[3 * NBUF : 4 * NBUF]
        wid = lax.axis_index("s") * NUM_CORES + lax.axis_index("c")
        base = wid * per_w
        pltpu.sync_copy(idx_hbm.at[wid], idx_v)

        for b in range(NBUF):
            pltpu.async_copy(table_hbm.at[idx_v.at[b]], gbufs[b], gsems[b])

        def outer(g, carry):
            for b in range(NBUF):
                j = g * NBUF + b
                pltpu.make_async_copy(
                    table_hbm.at[idx_v.at[j]], gbufs[b], gsems[b]
                ).wait()

                @pl.when(g > 0)
                def _wait_scatter():
                    pltpu.make_async_copy(
                        sbufs[b], out_hbm.at[pl.ds(base, chunk)], ssems[b]
                    ).wait()

                def row(i, c2):
                    for q in range(EMBED // LANES):
                        sl = pl.ds(q * LANES, LANES)
                        sbufs[b][i, sl] = gbufs[b][i, sl] * SCALE
                    return c2

                lax.fori_loop(0, chunk, row, 0, unroll=8)

                pltpu.async_copy(
                    sbufs[b], out_hbm.at[pl.ds(base + j * chunk, chunk)], ssems[b]
                )
                jn = j + NBUF

                @pl.when(jn < n_chunks)
                def _next_gather():
                    pltpu.async_copy(table_hbm.at[idx_v.at[jn]], gbufs[b], gsems[b])

            return carry

        lax.fori_loop(0, n_outer, outer, 0)
        for b in range(NBUF):
            pltpu.make_async_copy(
                sbufs[b], out_hbm.at[pl.ds(base, chunk)], ssems[b]
            ).wait()

    return k(table, idx3)


def kernel(input, table):
    b, l = input.shape
    n = b * l
    per_w = n // NUM_WORKERS
    n_chunks = per_w // CHUNK
    idx3 = input.reshape(NUM_WORKERS, n_chunks, CHUNK).astype(jnp.int32)
    out = _emb_lookup(table, idx3)
    return out.reshape(b, l, EMBED)
